# baseline (speedup 1.0000x reference)
import functools

import jax
import jax.numpy as jnp
from jax import lax
from jax.experimental import pallas as pl
from jax.experimental.pallas import tpu as pltpu
from jax.experimental.pallas import tpu_sc as plsc

_NC = 2
_NS = 16
_NW = _NC * _NS
_CH = 128


@functools.partial(jax.jit, static_argnames=("nchunk", "d"))
def _sc_gather(emb_weight, idx3, nchunk, d):
    total = _NW * nchunk * _CH

    mesh = plsc.VectorSubcoreMesh(core_axis_name="c", subcore_axis_name="s")

    @functools.partial(
        pl.kernel,
        out_type=jax.ShapeDtypeStruct((total, d), jnp.float32),
        mesh=mesh,
        scratch_types=[
            pltpu.VMEM((nchunk, _CH), jnp.int32),
            pltpu.VMEM((_CH, d), jnp.float32),
            pltpu.SemaphoreType.DMA,
        ],
        compiler_params=pltpu.CompilerParams(use_tc_tiling_on_sc=False),
    )
    def body(table_hbm, idx_hbm, out_hbm, idx_v, rows_v, sem):
        wid = lax.axis_index("s") * _NC + lax.axis_index("c")
        pltpu.sync_copy(idx_hbm.at[wid], idx_v)
        base = wid * (nchunk * _CH)

        def step(j, carry):
            pltpu.async_copy(table_hbm.at[idx_v.at[j]], rows_v, sem).wait()
            pltpu.sync_copy(rows_v, out_hbm.at[pl.ds(base + j * _CH, _CH)])
            return carry

        lax.fori_loop(0, nchunk, step, 0)

    return body(emb_weight, idx3)


def kernel(X, emb_weight):
    b, t = X.shape
    v, d = emb_weight.shape
    total = b * t
    assert total % (_NW * _CH) == 0
    nchunk = total // (_NW * _CH)
    idx3 = X.astype(jnp.int32).reshape(_NW, nchunk, _CH)
    out = _sc_gather(emb_weight, idx3, nchunk, d)
    return out.reshape(b, t, d)


# speedup vs baseline: 1.1096x; 1.1096x over previous
import functools

import jax
import jax.numpy as jnp
from jax import lax
from jax.experimental import pallas as pl
from jax.experimental.pallas import tpu as pltpu
from jax.experimental.pallas import tpu_sc as plsc

_NC = 2
_NS = 16
_NW = _NC * _NS
_CH = 128


_NBUF = 8


@functools.partial(jax.jit, static_argnames=("nchunk", "d"))
def _sc_gather(emb_weight, idx3, nchunk, d):
    total = _NW * nchunk * _CH
    ngroup = nchunk // _NBUF

    mesh = plsc.VectorSubcoreMesh(core_axis_name="c", subcore_axis_name="s")

    @functools.partial(
        pl.kernel,
        out_type=jax.ShapeDtypeStruct((total, d), jnp.float32),
        mesh=mesh,
        scratch_types=[
            pltpu.VMEM((nchunk, _CH), jnp.int32),
            pltpu.VMEM((_NBUF, _CH, d), jnp.float32),
            [pltpu.SemaphoreType.DMA] * _NBUF,
            [pltpu.SemaphoreType.DMA] * _NBUF,
        ],
        compiler_params=pltpu.CompilerParams(use_tc_tiling_on_sc=False),
    )
    def body(table_hbm, idx_hbm, out_hbm, idx_v, rows_v, gsem, ssem):
        wid = lax.axis_index("s") * _NC + lax.axis_index("c")
        pltpu.sync_copy(idx_hbm.at[wid], idx_v)
        base = wid * (nchunk * _CH)

        def group(g, carry):
            j0 = g * _NBUF
            for b in range(_NBUF):
                @pl.when(g > 0)
                def _wait_store():
                    pltpu.make_async_copy(
                        rows_v.at[b],
                        out_hbm.at[pl.ds(base, _CH)],
                        ssem[b],
                    ).wait()

                pltpu.async_copy(
                    table_hbm.at[idx_v.at[j0 + b]], rows_v.at[b], gsem[b]
                )
            for b in range(_NBUF):
                pltpu.make_async_copy(
                    table_hbm.at[idx_v.at[j0 + b]], rows_v.at[b], gsem[b]
                ).wait()
                pltpu.async_copy(
                    rows_v.at[b],
                    out_hbm.at[pl.ds(base + (j0 + b) * _CH, _CH)],
                    ssem[b],
                )
            return carry

        lax.fori_loop(0, ngroup, group, 0)
        for b in range(_NBUF):
            pltpu.make_async_copy(
                rows_v.at[b], out_hbm.at[pl.ds(base, _CH)], ssem[b]
            ).wait()

    return body(emb_weight, idx3)


def kernel(X, emb_weight):
    b, t = X.shape
    v, d = emb_weight.shape
    total = b * t
    assert total % (_NW * _CH) == 0
    nchunk = total // (_NW * _CH)
    idx3 = X.astype(jnp.int32).reshape(_NW, nchunk, _CH)
    out = _sc_gather(emb_weight, idx3, nchunk, d)
    return out.reshape(b, t, d)


# speedup vs baseline: 1.4787x; 1.3326x over previous
import functools

import jax
import jax.numpy as jnp
from jax import lax
from jax.experimental import pallas as pl
from jax.experimental.pallas import tpu as pltpu
from jax.experimental.pallas import tpu_sc as plsc

_NC = 2
_NS = 16
_NW = _NC * _NS
_CH = 128
_NBUF = 8


@functools.partial(jax.jit, static_argnames=("nchunk", "d"))
def _sc_gather(tbl, idx3, nchunk, d):
    total = _NW * nchunk * _CH
    ngroup = nchunk // _NBUF

    mesh = plsc.VectorSubcoreMesh(core_axis_name="c", subcore_axis_name="s")

    @functools.partial(
        pl.kernel,
        out_type=jax.ShapeDtypeStruct((total, 128), jnp.float32),
        mesh=mesh,
        scratch_types=[
            pltpu.VMEM((nchunk, _CH), jnp.int32),
            pltpu.VMEM((_NBUF, _CH, d), jnp.float32),
            [pltpu.SemaphoreType.DMA] * _NBUF,
            [pltpu.SemaphoreType.DMA] * _NBUF,
        ],
        compiler_params=pltpu.CompilerParams(use_tc_tiling_on_sc=False),
    )
    def body(table_hbm, idx_hbm, out_hbm, idx_v, rows_v, gsem, ssem):
        wid = lax.axis_index("s") * _NC + lax.axis_index("c")
        pltpu.sync_copy(idx_hbm.at[wid], idx_v)
        base = wid * (nchunk * _CH)

        def group(g, carry):
            j0 = g * _NBUF
            for b in range(_NBUF):
                @pl.when(g > 0)
                def _wait_store():
                    pltpu.make_async_copy(
                        rows_v.at[b],
                        out_hbm.at[pl.ds(base, _CH), pl.ds(0, d)],
                        ssem[b],
                    ).wait()

                pltpu.async_copy(
                    table_hbm.at[idx_v.at[j0 + b]], rows_v.at[b], gsem[b]
                )
            for b in range(_NBUF):
                pltpu.make_async_copy(
                    table_hbm.at[idx_v.at[j0 + b]], rows_v.at[b], gsem[b]
                ).wait()
                pltpu.async_copy(
                    rows_v.at[b],
                    out_hbm.at[pl.ds(base + (j0 + b) * _CH, _CH), pl.ds(0, d)],
                    ssem[b],
                )
            return carry

        lax.fori_loop(0, ngroup, group, 0)
        for b in range(_NBUF):
            pltpu.make_async_copy(
                rows_v.at[b], out_hbm.at[pl.ds(base, _CH), pl.ds(0, d)], ssem[b]
            ).wait()

    return body(tbl, idx3)


def kernel(X, emb_weight):
    b, t = X.shape
    v, d = emb_weight.shape
    total = b * t
    assert total % (_NW * _CH) == 0
    nchunk = total // (_NW * _CH)
    idx3 = X.astype(jnp.int32).reshape(_NW, nchunk, _CH)
    out128 = _sc_gather(emb_weight, idx3, nchunk, d)
    return out128.reshape(b, t, 128)[:, :, :d]


# speedup vs baseline: 1.4804x; 1.0012x over previous
import functools

import jax
import jax.numpy as jnp
from jax import lax
from jax.experimental import pallas as pl
from jax.experimental.pallas import tpu as pltpu
from jax.experimental.pallas import tpu_sc as plsc

_NC = 2
_NS = 16
_NW = _NC * _NS
_CH = 128
_NBUF = 8


@functools.partial(jax.jit, static_argnames=("nchunk", "d", "v"))
def _sc_gather(tbl1d, idx3, nchunk, d, v):
    tbl = tbl1d.reshape(v, d)
    total = _NW * nchunk * _CH
    ngroup = nchunk // _NBUF

    mesh = plsc.VectorSubcoreMesh(core_axis_name="c", subcore_axis_name="s")

    @functools.partial(
        pl.kernel,
        out_type=jax.ShapeDtypeStruct((total, 128), jnp.float32),
        mesh=mesh,
        scratch_types=[
            pltpu.VMEM((nchunk, _CH), jnp.int32),
            pltpu.VMEM((_NBUF, _CH, d), jnp.float32),
            [pltpu.SemaphoreType.DMA] * _NBUF,
            [pltpu.SemaphoreType.DMA] * _NBUF,
        ],
        compiler_params=pltpu.CompilerParams(use_tc_tiling_on_sc=False),
    )
    def body(table_hbm, idx_hbm, out_hbm, idx_v, rows_v, gsem, ssem):
        wid = lax.axis_index("s") * _NC + lax.axis_index("c")
        pltpu.sync_copy(idx_hbm.at[wid], idx_v)
        base = wid * (nchunk * _CH)

        def group(g, carry):
            j0 = g * _NBUF
            for b in range(_NBUF):
                @pl.when(g > 0)
                def _wait_store():
                    pltpu.make_async_copy(
                        rows_v.at[b],
                        out_hbm.at[pl.ds(base, _CH), pl.ds(0, d)],
                        ssem[b],
                    ).wait()

                pltpu.async_copy(
                    table_hbm.at[idx_v.at[j0 + b]], rows_v.at[b], gsem[b]
                )
            for b in range(_NBUF):
                pltpu.make_async_copy(
                    table_hbm.at[idx_v.at[j0 + b]], rows_v.at[b], gsem[b]
                ).wait()
                pltpu.async_copy(
                    rows_v.at[b],
                    out_hbm.at[pl.ds(base + (j0 + b) * _CH, _CH), pl.ds(0, d)],
                    ssem[b],
                )
            return carry

        lax.fori_loop(0, ngroup, group, 0)
        for b in range(_NBUF):
            pltpu.make_async_copy(
                rows_v.at[b], out_hbm.at[pl.ds(base, _CH), pl.ds(0, d)], ssem[b]
            ).wait()

    return body(tbl, idx3)


def kernel(X, emb_weight):
    b, t = X.shape
    v, d = emb_weight.shape
    total = b * t
    assert total % (_NW * _CH) == 0
    nchunk = total // (_NW * _CH)
    idx3 = X.astype(jnp.int32).reshape(_NW, nchunk, _CH)
    out128 = _sc_gather(emb_weight.reshape(-1), idx3, nchunk, d, v)
    return out128.reshape(b, t, 128)[:, :, :d]


# speedup vs baseline: 1.4820x; 1.0011x over previous
import functools

import jax
import jax.numpy as jnp
from jax import lax
from jax.experimental import pallas as pl
from jax.experimental.pallas import tpu as pltpu
from jax.experimental.pallas import tpu_sc as plsc

_NC = 2
_NS = 16
_NW = _NC * _NS
_CH = 128
_NBUF = 10


@functools.partial(jax.jit, static_argnames=("nchunk", "d", "v"))
def _sc_gather(tbl1d, idx3, nchunk, d, v):
    tbl = tbl1d.reshape(v, d)
    total = _NW * nchunk * _CH
    ngroup = nchunk // _NBUF

    mesh = plsc.VectorSubcoreMesh(core_axis_name="c", subcore_axis_name="s")

    @functools.partial(
        pl.kernel,
        out_type=jax.ShapeDtypeStruct((total, 128), jnp.float32),
        mesh=mesh,
        scratch_types=[
            pltpu.VMEM((nchunk, _CH), jnp.int32),
            pltpu.VMEM((_NBUF, _CH, d), jnp.float32),
            [pltpu.SemaphoreType.DMA] * _NBUF,
            [pltpu.SemaphoreType.DMA] * _NBUF,
        ],
        compiler_params=pltpu.CompilerParams(use_tc_tiling_on_sc=False),
    )
    def body(table_hbm, idx_hbm, out_hbm, idx_v, rows_v, gsem, ssem):
        wid = lax.axis_index("s") * _NC + lax.axis_index("c")
        pltpu.sync_copy(idx_hbm.at[wid], idx_v)
        base = wid * (nchunk * _CH)

        def group(g, carry):
            j0 = g * _NBUF
            for b in range(_NBUF):
                @pl.when(g > 0)
                def _wait_store():
                    pltpu.make_async_copy(
                        rows_v.at[b],
                        out_hbm.at[pl.ds(base, _CH), pl.ds(0, d)],
                        ssem[b],
                    ).wait()

                pltpu.async_copy(
                    table_hbm.at[idx_v.at[j0 + b]], rows_v.at[b], gsem[b]
                )
            for b in range(_NBUF):
                pltpu.make_async_copy(
                    table_hbm.at[idx_v.at[j0 + b]], rows_v.at[b], gsem[b]
                ).wait()
                pltpu.async_copy(
                    rows_v.at[b],
                    out_hbm.at[pl.ds(base + (j0 + b) * _CH, _CH), pl.ds(0, d)],
                    ssem[b],
                )
            return carry

        lax.fori_loop(0, ngroup, group, 0)
        for b in range(_NBUF):
            pltpu.make_async_copy(
                rows_v.at[b], out_hbm.at[pl.ds(base, _CH), pl.ds(0, d)], ssem[b]
            ).wait()

    return body(tbl, idx3)


def kernel(X, emb_weight):
    b, t = X.shape
    v, d = emb_weight.shape
    total = b * t
    assert total % (_NW * _CH) == 0
    nchunk = total // (_NW * _CH)
    idx3 = X.astype(jnp.int32).reshape(_NW, nchunk, _CH)
    out128 = _sc_gather(emb_weight.reshape(-1), idx3, nchunk, d, v)
    return out128.reshape(b, t, 128)[:, :, :d]


# speedup vs baseline: 1.7567x; 1.1853x over previous
import functools

import jax
import jax.numpy as jnp
from jax import lax
from jax.experimental import pallas as pl
from jax.experimental.pallas import tpu as pltpu
from jax.experimental.pallas import tpu_sc as plsc

_NC = 2
_NS = 16
_NW = _NC * _NS
_CH = 128
_NBUF = 10


_TB = 8192


def _tc_repack(embT, v, d):
    def kern(x_ref, i_ref, o_ref):
        o_ref[...] = jax.lax.dot_general(
            x_ref[...],
            i_ref[...],
            (((0,), (0,)), ((), ())),
            precision=jax.lax.Precision.HIGHEST,
        )

    ident2 = jnp.concatenate(
        [jnp.eye(d, dtype=jnp.float32)] * 2, axis=1
    )
    return pl.pallas_call(
        kern,
        grid=(pl.cdiv(v, _TB),),
        in_specs=[
            pl.BlockSpec((d, _TB), lambda c: (0, c)),
            pl.BlockSpec((d, 2 * d), lambda c: (0, 0)),
        ],
        out_specs=pl.BlockSpec((_TB, 2 * d), lambda c: (c, 0)),
        out_shape=jax.ShapeDtypeStruct((v, 2 * d), jnp.float32),
    )(embT, ident2)


@functools.partial(jax.jit, static_argnames=("nchunk", "d", "v"))
def _sc_gather(tbl1d, idx3, nchunk, d, v):
    tbl = tbl1d.reshape(v, d)
    total = _NW * nchunk * _CH
    ngroup = nchunk // _NBUF

    mesh = plsc.VectorSubcoreMesh(core_axis_name="c", subcore_axis_name="s")

    @functools.partial(
        pl.kernel,
        out_type=jax.ShapeDtypeStruct((total, 128), jnp.float32),
        mesh=mesh,
        scratch_types=[
            pltpu.VMEM((nchunk, _CH), jnp.int32),
            pltpu.VMEM((_NBUF, _CH, d), jnp.float32),
            [pltpu.SemaphoreType.DMA] * _NBUF,
            [pltpu.SemaphoreType.DMA] * _NBUF,
        ],
        compiler_params=pltpu.CompilerParams(use_tc_tiling_on_sc=False),
    )
    def body(table_hbm, idx_hbm, out_hbm, idx_v, rows_v, gsem, ssem):
        wid = lax.axis_index("s") * _NC + lax.axis_index("c")
        pltpu.sync_copy(idx_hbm.at[wid], idx_v)
        base = wid * (nchunk * _CH)

        def group(g, carry):
            j0 = g * _NBUF
            for b in range(_NBUF):
                @pl.when(g > 0)
                def _wait_store():
                    pltpu.make_async_copy(
                        rows_v.at[b],
                        out_hbm.at[pl.ds(base, _CH), pl.ds(0, d)],
                        ssem[b],
                    ).wait()

                pltpu.async_copy(
                    table_hbm.at[idx_v.at[j0 + b]], rows_v.at[b], gsem[b]
                )
            for b in range(_NBUF):
                pltpu.make_async_copy(
                    table_hbm.at[idx_v.at[j0 + b]], rows_v.at[b], gsem[b]
                ).wait()
                pltpu.async_copy(
                    rows_v.at[b],
                    out_hbm.at[pl.ds(base + (j0 + b) * _CH, _CH), pl.ds(0, d)],
                    ssem[b],
                )
            return carry

        lax.fori_loop(0, ngroup, group, 0)
        for b in range(_NBUF):
            pltpu.make_async_copy(
                rows_v.at[b], out_hbm.at[pl.ds(base, _CH), pl.ds(0, d)], ssem[b]
            ).wait()

    return body(tbl, idx3)


def kernel(X, emb_weight):
    b, t = X.shape
    v, d = emb_weight.shape
    total = b * t
    assert total % (_NW * _CH) == 0
    nchunk = total // (_NW * _CH)
    idx3 = (X.astype(jnp.int32) * 2).reshape(_NW, nchunk, _CH)
    tbl_lin = _tc_repack(emb_weight.T, v, d)
    out128 = _sc_gather(tbl_lin.reshape(-1), idx3, nchunk, d, 2 * v)
    return out128.reshape(b, t, 128)[:, :, :d]


# speedup vs baseline: 1.7791x; 1.0128x over previous
import functools

import jax
import jax.numpy as jnp
from jax import lax
from jax.experimental import pallas as pl
from jax.experimental.pallas import tpu as pltpu
from jax.experimental.pallas import tpu_sc as plsc

_NC = 2
_NS = 16
_NW = _NC * _NS
_CH = 128
_NBUF = 10


_TB = 12288


def _tc_repack(embT, v, d):
    def kern(x_ref, i_ref, o_ref):
        o_ref[...] = jax.lax.dot_general(
            x_ref[...],
            i_ref[...],
            (((0,), (0,)), ((), ())),
            precision=jax.lax.Precision.HIGHEST,
        )

    ident2 = jnp.concatenate(
        [jnp.eye(d, dtype=jnp.float32)] * 2, axis=1
    )
    return pl.pallas_call(
        kern,
        grid=(pl.cdiv(v, _TB),),
        in_specs=[
            pl.BlockSpec((d, _TB), lambda c: (0, c)),
            pl.BlockSpec((d, 2 * d), lambda c: (0, 0)),
        ],
        out_specs=pl.BlockSpec((_TB, 2 * d), lambda c: (c, 0)),
        out_shape=jax.ShapeDtypeStruct((v, 2 * d), jnp.float32),
    )(embT, ident2)


@functools.partial(jax.jit, static_argnames=("nchunk", "d", "v"))
def _sc_gather(tbl1d, idx3, nchunk, d, v):
    tbl = tbl1d.reshape(v, d)
    total = _NW * nchunk * _CH
    ngroup = nchunk // _NBUF

    mesh = plsc.VectorSubcoreMesh(core_axis_name="c", subcore_axis_name="s")

    @functools.partial(
        pl.kernel,
        out_type=jax.ShapeDtypeStruct((total, 128), jnp.float32),
        mesh=mesh,
        scratch_types=[
            pltpu.VMEM((nchunk, _CH), jnp.int32),
            pltpu.VMEM((_NBUF, _CH, d), jnp.float32),
            [pltpu.SemaphoreType.DMA] * _NBUF,
            [pltpu.SemaphoreType.DMA] * _NBUF,
        ],
        compiler_params=pltpu.CompilerParams(use_tc_tiling_on_sc=False),
    )
    def body(table_hbm, idx_hbm, out_hbm, idx_v, rows_v, gsem, ssem):
        wid = lax.axis_index("s") * _NC + lax.axis_index("c")
        pltpu.sync_copy(idx_hbm.at[wid], idx_v)
        base = wid * (nchunk * _CH)

        def group(g, carry):
            j0 = g * _NBUF
            for b in range(_NBUF):
                @pl.when(g > 0)
                def _wait_store():
                    pltpu.make_async_copy(
                        rows_v.at[b],
                        out_hbm.at[pl.ds(base, _CH), pl.ds(0, d)],
                        ssem[b],
                    ).wait()

                pltpu.async_copy(
                    table_hbm.at[idx_v.at[j0 + b]], rows_v.at[b], gsem[b]
                )
            for b in range(_NBUF):
                pltpu.make_async_copy(
                    table_hbm.at[idx_v.at[j0 + b]], rows_v.at[b], gsem[b]
                ).wait()
                pltpu.async_copy(
                    rows_v.at[b],
                    out_hbm.at[pl.ds(base + (j0 + b) * _CH, _CH), pl.ds(0, d)],
                    ssem[b],
                )
            return carry

        lax.fori_loop(0, ngroup, group, 0)
        for b in range(_NBUF):
            pltpu.make_async_copy(
                rows_v.at[b], out_hbm.at[pl.ds(base, _CH), pl.ds(0, d)], ssem[b]
            ).wait()

    return body(tbl, idx3)


def kernel(X, emb_weight):
    b, t = X.shape
    v, d = emb_weight.shape
    total = b * t
    assert total % (_NW * _CH) == 0
    nchunk = total // (_NW * _CH)
    idx3 = (X.astype(jnp.int32) * 2).reshape(_NW, nchunk, _CH)
    tbl_lin = _tc_repack(emb_weight.T, v, d)
    out128 = _sc_gather(tbl_lin.reshape(-1), idx3, nchunk, d, 2 * v)
    return out128.reshape(b, t, 128)[:, :, :d]
